# baseline (speedup 1.0000x reference)
import functools
import jax
import jax.numpy as jnp
from jax import lax
from jax.experimental import pallas as pl
from jax.experimental.pallas import tpu as pltpu
from jax.experimental.pallas import tpu_sc as plsc

N = 100000
E = 1600000
NPAD = 100352
NB = 49
BN = 2048
NC = 2
NS = 16
NW = NC * NS
EPAD = 1638400
ROWS = EPAD // 128
RPW = ROWS // NW
CHUNK_ROWS = 8
NCHUNKS = RPW // CHUNK_ROWS
RPW0 = 544
RPW1 = 256
EPW = RPW * 128
SLAB = NPAD // NS
NSLOT = 4


def _tc_node_body(nc, x_ref, w_ref, as_ref, ad_ref, *out_refs):
    h = jnp.dot(x_ref[...], w_ref[...], preferred_element_type=jnp.float32)
    asrc_ref, adst_ref = out_refs[nc], out_refs[nc + 1]
    asrc_ref[...] = jnp.sum(h * as_ref[0], axis=1)
    adst_ref[...] = jnp.sum(h * ad_ref[0], axis=1)
    for c in range(nc):
        out_refs[c][...] = h[:, c * 16:(c + 1) * 16]


def _tc_node(x, W, att_src, att_dst, nc):
    fin = x.shape[1]
    as_r = att_src.reshape(1, nc * 16)
    ad_r = att_dst.reshape(1, nc * 16)
    return pl.pallas_call(
        functools.partial(_tc_node_body, nc),
        grid=(NB,),
        in_specs=[
            pl.BlockSpec((BN, fin), lambda i: (i, 0)),
            pl.BlockSpec((fin, nc * 16), lambda i: (0, 0)),
            pl.BlockSpec((1, nc * 16), lambda i: (0, 0)),
            pl.BlockSpec((1, nc * 16), lambda i: (0, 0)),
        ],
        out_specs=[pl.BlockSpec((BN, 16), lambda i: (i, 0))] * nc
        + [pl.BlockSpec((BN,), lambda i: (i,))] * 2,
        out_shape=[jax.ShapeDtypeStruct((NPAD, 16), jnp.float32)] * nc
        + [jax.ShapeDtypeStruct((NPAD,), jnp.float32)] * 2,
    )(x, W, as_r, ad_r)


def _tc_fin_body(nc, nout, acc_ref, den_ref, asrc_ref, adst_ref, b_ref,
                 *hc_and_out):
    hc_refs = hc_and_out[:nc]
    out_ref = hc_and_out[nc]
    a = asrc_ref[...] + adst_ref[...]
    ex_self = jnp.exp(jnp.where(a > 0, a, 0.2 * a))
    den = den_ref[0] + den_ref[1] + ex_self + 1e-16
    pieces = [acc_ref[0, c] + acc_ref[1, c] + ex_self[:, None] * hc_refs[c][...]
              for c in range(nc)]
    acc = jnp.concatenate(pieces, axis=1)
    out_ref[...] = jnp.maximum(acc / den[:, None] + b_ref[...], 0.0)


def _tc_finalize(acc, den, asrc, adst, hc_list, bias, nc, nout):
    b_r = bias.reshape(1, nc * 16)
    return pl.pallas_call(
        functools.partial(_tc_fin_body, nc, nout),
        grid=(NB,),
        in_specs=[
            pl.BlockSpec((NC, nc, BN, 16), lambda i: (0, 0, i, 0)),
            pl.BlockSpec((NC, BN), lambda i: (0, i)),
            pl.BlockSpec((BN,), lambda i: (i,)),
            pl.BlockSpec((BN,), lambda i: (i,)),
            pl.BlockSpec((1, nc * 16), lambda i: (0, 0)),
        ] + [pl.BlockSpec((BN, 16), lambda i: (i, 0))] * nc,
        out_specs=pl.BlockSpec((BN, nc * 16), lambda i: (i, 0)),
        out_shape=jax.ShapeDtypeStruct((nout, nc * 16), jnp.float32),
    )(acc, den, asrc, adst, b_r, *hc_list)


_BCAST_DN = lax.GatherDimensionNumbers(
    offset_dims=(), collapsed_slice_dims=(0,), start_index_map=(0,))


def _lane_bcast(vec16, lane):
    return lax.gather(vec16, jnp.full((16, 1), lane, jnp.int32), _BCAST_DN,
                      (1,), mode=lax.GatherScatterMode.PROMISE_IN_BOUNDS)


def _sc_edges_body(nc, src_hbm, dst_hbm, asrc_hbm, adst_hbm,
                   zeros1_hbm, zeros2_hbm, *rest):
    hc_hbms = rest[:nc]
    ex_hbm, den_hbm, acc_hbm = rest[nc:nc + 3]
    (src_v, dst_v, asrc_b, adst_b, rows_b, ex_v,
     den_sh, acc_sh, g1, g2, ss, st) = rest[nc + 3:]
    c = lax.axis_index("c")
    s = lax.axis_index("s")
    base_r = jnp.where(c == 0, s * RPW0, NS * RPW0 + s * RPW1)
    nch = jnp.where(c == 0, RPW0 // CHUNK_ROWS, RPW1 // CHUNK_ROWS)

    pltpu.sync_copy(zeros1_hbm.at[pl.ds(s * SLAB, SLAB)],
                    den_sh.at[pl.ds(s * SLAB, SLAB)])
    plsc.subcore_barrier()

    def fire_a(i, r, slot):
        pltpu.async_copy(asrc_hbm.at[src_v.at[r]], asrc_b.at[slot],
                         g1.at[slot])
        pltpu.async_copy(adst_hbm.at[dst_v.at[r]], adst_b.at[slot],
                         g2.at[slot])

    def chunk_a(i, _):
        r0 = base_r + i * CHUNK_ROWS
        pltpu.sync_copy(src_hbm.at[pl.ds(r0, CHUNK_ROWS)], src_v)
        pltpu.sync_copy(dst_hbm.at[pl.ds(r0, CHUNK_ROWS)], dst_v)
        for r in range(2):
            fire_a(i, r, r % NSLOT)
        for r in range(CHUNK_ROWS):
            slot = r % NSLOT
            if r + 2 < CHUNK_ROWS:
                fire_a(i, r + 2, (r + 2) % NSLOT)
            pltpu.make_async_copy(asrc_hbm.at[src_v.at[r]],
                                  asrc_b.at[slot], g1.at[slot]).wait()
            pltpu.make_async_copy(adst_hbm.at[dst_v.at[r]],
                                  adst_b.at[slot], g2.at[slot]).wait()
            eb = r * 128

            def comp(k, _):
                a = (asrc_b[slot, pl.ds(k * 16, 16)]
                     + adst_b[slot, pl.ds(k * 16, 16)])
                ex_v[pl.ds(eb + k * 16, 16)] = \
                    jnp.exp(jnp.where(a > 0, a, 0.2 * a))
                return 0

            lax.fori_loop(0, 8, comp, 0)
            pltpu.async_copy(ex_v.at[pl.ds(eb, 128)],
                             den_sh.at[dst_v.at[r]], ss.at[slot], add=True)
        pltpu.async_copy(ex_v, ex_hbm.at[pl.ds(r0 * 128,
                                               CHUNK_ROWS * 128)], st)
        for r in range(CHUNK_ROWS):
            pltpu.make_async_copy(ex_v.at[pl.ds(0, 128)],
                                  den_sh.at[dst_v.at[0]],
                                  ss.at[r % NSLOT]).wait()
        pltpu.make_async_copy(ex_v, ex_hbm.at[pl.ds(0, CHUNK_ROWS * 128)],
                              st).wait()
        return 0

    lax.fori_loop(0, nch, chunk_a, 0)
    plsc.subcore_barrier()
    pltpu.sync_copy(den_sh.at[pl.ds(s * SLAB, SLAB)],
                    den_hbm.at[c, pl.ds(s * SLAB, SLAB)])

    for fc in range(nc):
        pltpu.sync_copy(zeros2_hbm.at[pl.ds(s * SLAB, SLAB)],
                        acc_sh.at[pl.ds(s * SLAB, SLAB)])
        plsc.subcore_barrier()

        def fire_b(r, slot):
            pltpu.async_copy(hc_hbms[fc].at[src_v.at[r]],
                             rows_b.at[slot], g1.at[slot])

        def chunk_b(i, _):
            r0 = base_r + i * CHUNK_ROWS
            pltpu.sync_copy(src_hbm.at[pl.ds(r0, CHUNK_ROWS)], src_v)
            pltpu.sync_copy(dst_hbm.at[pl.ds(r0, CHUNK_ROWS)], dst_v)
            pltpu.sync_copy(ex_hbm.at[pl.ds(r0 * 128, CHUNK_ROWS * 128)],
                            ex_v)
            for r in range(2):
                fire_b(r, r % NSLOT)
            for r in range(CHUNK_ROWS):
                slot = r % NSLOT
                if r + 2 < CHUNK_ROWS:
                    if r >= 2:
                        pltpu.make_async_copy(
                            rows_b.at[(r + 2) % NSLOT],
                            acc_sh.at[dst_v.at[r]],
                            ss.at[(r + 2) % NSLOT]).wait()
                    fire_b(r + 2, (r + 2) % NSLOT)
                pltpu.make_async_copy(hc_hbms[fc].at[src_v.at[r]],
                                      rows_b.at[slot], g1.at[slot]).wait()
                eb = r * 128

                def comp(g, _):
                    coefs = ex_v[pl.ds(eb + g * 16, 16)]
                    for l in range(16):
                        e = g * 16 + l
                        rows_b[slot, e, :] = \
                            rows_b[slot, e, :] * _lane_bcast(coefs, l)
                    return 0

                lax.fori_loop(0, 8, comp, 0)
                pltpu.async_copy(rows_b.at[slot], acc_sh.at[dst_v.at[r]],
                                 ss.at[slot], add=True)
            for slot in range(NSLOT):
                pltpu.make_async_copy(rows_b.at[slot],
                                      acc_sh.at[dst_v.at[0]],
                                      ss.at[slot]).wait()
            return 0

        lax.fori_loop(0, nch, chunk_b, 0)
        plsc.subcore_barrier()
        pltpu.sync_copy(acc_sh.at[pl.ds(s * SLAB, SLAB)],
                        acc_hbm.at[c, fc, pl.ds(s * SLAB, SLAB)])
        plsc.subcore_barrier()


def _sc_edges(src, dst, asrc, adst, hc_list, zeros1d, zeros2d, nc):
    mesh = plsc.VectorSubcoreMesh(core_axis_name="c", subcore_axis_name="s")
    return pl.kernel(
        functools.partial(_sc_edges_body, nc),
        out_type=[
            jax.ShapeDtypeStruct((EPAD,), jnp.float32),
            jax.ShapeDtypeStruct((NC, NPAD), jnp.float32),
            jax.ShapeDtypeStruct((NC, nc, NPAD, 16), jnp.float32),
        ],
        mesh=mesh,
        compiler_params=pltpu.CompilerParams(use_tc_tiling_on_sc=False),
        scratch_types=[
            pltpu.VMEM((CHUNK_ROWS, 128), jnp.int32),
            pltpu.VMEM((CHUNK_ROWS, 128), jnp.int32),
            pltpu.VMEM((NSLOT, 128), jnp.float32),
            pltpu.VMEM((NSLOT, 128), jnp.float32),
            pltpu.VMEM((NSLOT, 128, 16), jnp.float32),
            pltpu.VMEM((CHUNK_ROWS * 128,), jnp.float32),
            pltpu.VMEM_SHARED((NPAD,), jnp.float32),
            pltpu.VMEM_SHARED((NPAD, 16), jnp.float32),
            pltpu.SemaphoreType.DMA((NSLOT,)),
            pltpu.SemaphoreType.DMA((NSLOT,)),
            pltpu.SemaphoreType.DMA((NSLOT,)),
            pltpu.SemaphoreType.DMA,
        ],
    )(src, dst, asrc, adst, zeros1d, zeros2d, *hc_list)


def _layer(x_p, src, dst, zeros1d, zeros2d, W, att_src, att_dst, bias, nc,
           nout):
    outs = _tc_node(x_p, W, att_src, att_dst, nc)
    hc_list = outs[:nc]
    asrc, adst = outs[nc], outs[nc + 1]
    _, den, acc = _sc_edges(src, dst, asrc, adst, hc_list,
                            zeros1d, zeros2d, nc)
    return _tc_finalize(acc, den, asrc, adst, hc_list, bias, nc, nout)


def kernel(x, edge_index, W1, att_src1, att_dst1, b1,
           W2, att_src2, att_dst2, b2):
    x_p = jnp.pad(x, ((0, NPAD - N), (0, 0)))
    src = edge_index[0].astype(jnp.int32)
    dst = edge_index[1].astype(jnp.int32)
    pad_idx = jnp.full((EPAD - E,), NPAD - 1, jnp.int32)
    src = jnp.concatenate([src, pad_idx]).reshape(ROWS, 128)
    dst = jnp.concatenate([dst, pad_idx]).reshape(ROWS, 128)
    zeros1d = jnp.zeros((NPAD,), jnp.float32)
    zeros2d = jnp.zeros((NPAD, 16), jnp.float32)

    h1 = _layer(x_p, src, dst, zeros1d, zeros2d, W1, att_src1, att_dst1,
                b1, 1, NPAD)
    h2 = _layer(h1, src, dst, zeros1d, zeros2d, W2, att_src2, att_dst2,
                b2, 4, N)
    return h2


# speedup vs baseline: 1.0733x; 1.0733x over previous
import functools
import jax
import jax.numpy as jnp
from jax import lax
from jax.experimental import pallas as pl
from jax.experimental.pallas import tpu as pltpu
from jax.experimental.pallas import tpu_sc as plsc

N = 100000
E = 1600000
NPAD = 100352
NB = 49
BN = 2048
NC = 2
NS = 16
NW = NC * NS
EPAD = 1638400
ROWS = EPAD // 128
RPW = ROWS // NW
CHUNK_ROWS = 16
NCHUNKS = RPW // CHUNK_ROWS
RPW0 = 512
RPW1 = 288
EPW = RPW * 128
SLAB = NPAD // NS
NSLOT = 4


def _tc_node_body(nc, x_ref, w_ref, as_ref, ad_ref, *out_refs):
    h = jnp.dot(x_ref[...], w_ref[...], preferred_element_type=jnp.float32)
    asrc_ref, adst_ref = out_refs[nc], out_refs[nc + 1]
    asrc_ref[...] = jnp.sum(h * as_ref[0], axis=1)
    adst_ref[...] = jnp.sum(h * ad_ref[0], axis=1)
    for c in range(nc):
        out_refs[c][...] = h[:, c * 16:(c + 1) * 16]


def _tc_node(x, W, att_src, att_dst, nc):
    fin = x.shape[1]
    as_r = att_src.reshape(1, nc * 16)
    ad_r = att_dst.reshape(1, nc * 16)
    return pl.pallas_call(
        functools.partial(_tc_node_body, nc),
        grid=(NB,),
        in_specs=[
            pl.BlockSpec((BN, fin), lambda i: (i, 0)),
            pl.BlockSpec((fin, nc * 16), lambda i: (0, 0)),
            pl.BlockSpec((1, nc * 16), lambda i: (0, 0)),
            pl.BlockSpec((1, nc * 16), lambda i: (0, 0)),
        ],
        out_specs=[pl.BlockSpec((BN, 16), lambda i: (i, 0))] * nc
        + [pl.BlockSpec((BN,), lambda i: (i,))] * 2,
        out_shape=[jax.ShapeDtypeStruct((NPAD, 16), jnp.float32)] * nc
        + [jax.ShapeDtypeStruct((NPAD,), jnp.float32)] * 2,
    )(x, W, as_r, ad_r)


def _tc_fin_body(nc, nout, acc_ref, den_ref, asrc_ref, adst_ref, b_ref,
                 *hc_and_out):
    hc_refs = hc_and_out[:nc]
    out_ref = hc_and_out[nc]
    a = asrc_ref[...] + adst_ref[...]
    ex_self = jnp.exp(jnp.where(a > 0, a, 0.2 * a))
    den = den_ref[0] + den_ref[1] + ex_self + 1e-16
    pieces = [acc_ref[0, c] + acc_ref[1, c] + ex_self[:, None] * hc_refs[c][...]
              for c in range(nc)]
    acc = jnp.concatenate(pieces, axis=1)
    out_ref[...] = jnp.maximum(acc / den[:, None] + b_ref[...], 0.0)


def _tc_finalize(acc, den, asrc, adst, hc_list, bias, nc, nout):
    b_r = bias.reshape(1, nc * 16)
    return pl.pallas_call(
        functools.partial(_tc_fin_body, nc, nout),
        grid=(NB,),
        in_specs=[
            pl.BlockSpec((NC, nc, BN, 16), lambda i: (0, 0, i, 0)),
            pl.BlockSpec((NC, BN), lambda i: (0, i)),
            pl.BlockSpec((BN,), lambda i: (i,)),
            pl.BlockSpec((BN,), lambda i: (i,)),
            pl.BlockSpec((1, nc * 16), lambda i: (0, 0)),
        ] + [pl.BlockSpec((BN, 16), lambda i: (i, 0))] * nc,
        out_specs=pl.BlockSpec((BN, nc * 16), lambda i: (i, 0)),
        out_shape=jax.ShapeDtypeStruct((nout, nc * 16), jnp.float32),
    )(acc, den, asrc, adst, b_r, *hc_list)


_BCAST_DN = lax.GatherDimensionNumbers(
    offset_dims=(), collapsed_slice_dims=(0,), start_index_map=(0,))


def _lane_bcast(vec16, lane):
    return lax.gather(vec16, jnp.full((16, 1), lane, jnp.int32), _BCAST_DN,
                      (1,), mode=lax.GatherScatterMode.PROMISE_IN_BOUNDS)


def _sc_edges_body(nc, src_hbm, dst_hbm, asrc_hbm, adst_hbm,
                   zeros1_hbm, zeros2_hbm, *rest):
    hc_hbms = rest[:nc]
    ex_hbm, den_hbm, acc_hbm = rest[nc:nc + 3]
    (src_v, dst_v, asrc_b, adst_b, rows_b, ex_v,
     den_sh, acc_sh, g1, g2, ss, st) = rest[nc + 3:]
    c = lax.axis_index("c")
    s = lax.axis_index("s")
    base_r = jnp.where(c == 0, s * RPW0, NS * RPW0 + s * RPW1)
    nch = jnp.where(c == 0, RPW0 // CHUNK_ROWS, RPW1 // CHUNK_ROWS)

    pltpu.sync_copy(zeros1_hbm.at[pl.ds(s * SLAB, SLAB)],
                    den_sh.at[pl.ds(s * SLAB, SLAB)])
    plsc.subcore_barrier()

    def fire_a(i, r, slot):
        pltpu.async_copy(asrc_hbm.at[src_v.at[r]], asrc_b.at[slot],
                         g1.at[slot])
        pltpu.async_copy(adst_hbm.at[dst_v.at[r]], adst_b.at[slot],
                         g2.at[slot])

    def chunk_a(i, _):
        r0 = base_r + i * CHUNK_ROWS
        pltpu.sync_copy(src_hbm.at[pl.ds(r0, CHUNK_ROWS)], src_v)
        pltpu.sync_copy(dst_hbm.at[pl.ds(r0, CHUNK_ROWS)], dst_v)
        for r in range(2):
            fire_a(i, r, r % NSLOT)
        for r in range(CHUNK_ROWS):
            slot = r % NSLOT
            if r + 2 < CHUNK_ROWS:
                fire_a(i, r + 2, (r + 2) % NSLOT)
            pltpu.make_async_copy(asrc_hbm.at[src_v.at[r]],
                                  asrc_b.at[slot], g1.at[slot]).wait()
            pltpu.make_async_copy(adst_hbm.at[dst_v.at[r]],
                                  adst_b.at[slot], g2.at[slot]).wait()
            eb = r * 128

            def comp(k, _):
                a = (asrc_b[slot, pl.ds(k * 16, 16)]
                     + adst_b[slot, pl.ds(k * 16, 16)])
                ex_v[pl.ds(eb + k * 16, 16)] = \
                    jnp.exp(jnp.where(a > 0, a, 0.2 * a))
                return 0

            lax.fori_loop(0, 8, comp, 0)
            pltpu.async_copy(ex_v.at[pl.ds(eb, 128)],
                             den_sh.at[dst_v.at[r]], ss.at[slot], add=True)
        pltpu.async_copy(ex_v, ex_hbm.at[pl.ds(r0 * 128,
                                               CHUNK_ROWS * 128)], st)
        for r in range(CHUNK_ROWS):
            pltpu.make_async_copy(ex_v.at[pl.ds(0, 128)],
                                  den_sh.at[dst_v.at[0]],
                                  ss.at[r % NSLOT]).wait()
        pltpu.make_async_copy(ex_v, ex_hbm.at[pl.ds(0, CHUNK_ROWS * 128)],
                              st).wait()
        return 0

    lax.fori_loop(0, nch, chunk_a, 0)
    plsc.subcore_barrier()
    pltpu.sync_copy(den_sh.at[pl.ds(s * SLAB, SLAB)],
                    den_hbm.at[c, pl.ds(s * SLAB, SLAB)])

    for fc in range(nc):
        pltpu.sync_copy(zeros2_hbm.at[pl.ds(s * SLAB, SLAB)],
                        acc_sh.at[pl.ds(s * SLAB, SLAB)])
        plsc.subcore_barrier()

        def fire_b(r, slot):
            pltpu.async_copy(hc_hbms[fc].at[src_v.at[r]],
                             rows_b.at[slot], g1.at[slot])

        def chunk_b(i, _):
            r0 = base_r + i * CHUNK_ROWS
            pltpu.sync_copy(src_hbm.at[pl.ds(r0, CHUNK_ROWS)], src_v)
            pltpu.sync_copy(dst_hbm.at[pl.ds(r0, CHUNK_ROWS)], dst_v)
            pltpu.sync_copy(ex_hbm.at[pl.ds(r0 * 128, CHUNK_ROWS * 128)],
                            ex_v)
            for r in range(2):
                fire_b(r, r % NSLOT)
            for r in range(CHUNK_ROWS):
                slot = r % NSLOT
                if r + 2 < CHUNK_ROWS:
                    if r >= 2:
                        pltpu.make_async_copy(
                            rows_b.at[(r + 2) % NSLOT],
                            acc_sh.at[dst_v.at[r]],
                            ss.at[(r + 2) % NSLOT]).wait()
                    fire_b(r + 2, (r + 2) % NSLOT)
                pltpu.make_async_copy(hc_hbms[fc].at[src_v.at[r]],
                                      rows_b.at[slot], g1.at[slot]).wait()
                eb = r * 128

                def comp(g, _):
                    coefs = ex_v[pl.ds(eb + g * 16, 16)]
                    for l in range(16):
                        e = g * 16 + l
                        rows_b[slot, e, :] = \
                            rows_b[slot, e, :] * _lane_bcast(coefs, l)
                    return 0

                lax.fori_loop(0, 8, comp, 0)
                pltpu.async_copy(rows_b.at[slot], acc_sh.at[dst_v.at[r]],
                                 ss.at[slot], add=True)
            for slot in range(NSLOT):
                pltpu.make_async_copy(rows_b.at[slot],
                                      acc_sh.at[dst_v.at[0]],
                                      ss.at[slot]).wait()
            return 0

        lax.fori_loop(0, nch, chunk_b, 0)
        plsc.subcore_barrier()
        pltpu.sync_copy(acc_sh.at[pl.ds(s * SLAB, SLAB)],
                        acc_hbm.at[c, fc, pl.ds(s * SLAB, SLAB)])
        plsc.subcore_barrier()


def _sc_edges(src, dst, asrc, adst, hc_list, zeros1d, zeros2d, nc):
    mesh = plsc.VectorSubcoreMesh(core_axis_name="c", subcore_axis_name="s")
    return pl.kernel(
        functools.partial(_sc_edges_body, nc),
        out_type=[
            jax.ShapeDtypeStruct((EPAD,), jnp.float32),
            jax.ShapeDtypeStruct((NC, NPAD), jnp.float32),
            jax.ShapeDtypeStruct((NC, nc, NPAD, 16), jnp.float32),
        ],
        mesh=mesh,
        compiler_params=pltpu.CompilerParams(use_tc_tiling_on_sc=False),
        scratch_types=[
            pltpu.VMEM((CHUNK_ROWS, 128), jnp.int32),
            pltpu.VMEM((CHUNK_ROWS, 128), jnp.int32),
            pltpu.VMEM((NSLOT, 128), jnp.float32),
            pltpu.VMEM((NSLOT, 128), jnp.float32),
            pltpu.VMEM((NSLOT, 128, 16), jnp.float32),
            pltpu.VMEM((CHUNK_ROWS * 128,), jnp.float32),
            pltpu.VMEM_SHARED((NPAD,), jnp.float32),
            pltpu.VMEM_SHARED((NPAD, 16), jnp.float32),
            pltpu.SemaphoreType.DMA((NSLOT,)),
            pltpu.SemaphoreType.DMA((NSLOT,)),
            pltpu.SemaphoreType.DMA((NSLOT,)),
            pltpu.SemaphoreType.DMA,
        ],
    )(src, dst, asrc, adst, zeros1d, zeros2d, *hc_list)


def _layer(x_p, src, dst, zeros1d, zeros2d, W, att_src, att_dst, bias, nc,
           nout):
    outs = _tc_node(x_p, W, att_src, att_dst, nc)
    hc_list = outs[:nc]
    asrc, adst = outs[nc], outs[nc + 1]
    _, den, acc = _sc_edges(src, dst, asrc, adst, hc_list,
                            zeros1d, zeros2d, nc)
    return _tc_finalize(acc, den, asrc, adst, hc_list, bias, nc, nout)


def kernel(x, edge_index, W1, att_src1, att_dst1, b1,
           W2, att_src2, att_dst2, b2):
    x_p = jnp.pad(x, ((0, NPAD - N), (0, 0)))
    src = edge_index[0].astype(jnp.int32)
    dst = edge_index[1].astype(jnp.int32)
    pad_idx = jnp.full((EPAD - E,), NPAD - 1, jnp.int32)
    src = jnp.concatenate([src, pad_idx]).reshape(ROWS, 128)
    dst = jnp.concatenate([dst, pad_idx]).reshape(ROWS, 128)
    zeros1d = jnp.zeros((NPAD,), jnp.float32)
    zeros2d = jnp.zeros((NPAD, 16), jnp.float32)

    h1 = _layer(x_p, src, dst, zeros1d, zeros2d, W1, att_src1, att_dst1,
                b1, 1, NPAD)
    h2 = _layer(h1, src, dst, zeros1d, zeros2d, W2, att_src2, att_dst2,
                b2, 4, N)
    return h2
